# Initial kernel scaffold; baseline (speedup 1.0000x reference)
import jax
import jax.numpy as jnp
from jax.experimental import pallas as pl


def kernel(x, edge_index, W1, b1, W2, b2, W3, b3, Wl, bl):
    raise NotImplementedError("write your pallas kernel here")


# speedup vs baseline: 8.2578x; 8.2578x over previous
import functools

import jax
import jax.numpy as jnp
from jax import lax
from jax.experimental import pallas as pl
from jax.experimental.pallas import tpu as pltpu
from jax.experimental.pallas import tpu_sc as plsc

N = 10000
NP = 10240
E = 160000
CH = 128
NSUB = 16
NCHUNK = 79
EP = NSUB * NCHUNK * CH
RB = 512
GRID = NP // RB
RZ = NP // NSUB

_MESH = plsc.VectorSubcoreMesh(core_axis_name="c", subcore_axis_name="s")


def _make_deg_kernel():
    @functools.partial(
        pl.kernel,
        out_type=jax.ShapeDtypeStruct((2, NP, 128), jnp.float32),
        mesh=_MESH,
        scratch_types=[
            pltpu.VMEM((NCHUNK, CH), jnp.int32),
            pltpu.VMEM((CH, 128), jnp.float32),
            pltpu.VMEM_SHARED((NP, 128), jnp.float32),
        ],
    )
    def deg_kernel(dst_hbm, ones_hbm, zeros_hbm, out_hbm,
                   dst_v, ones_v, acc):
        c = lax.axis_index("c")
        s = lax.axis_index("s")
        pltpu.sync_copy(dst_hbm.at[s], dst_v)
        pltpu.sync_copy(ones_hbm, ones_v)
        pltpu.sync_copy(zeros_hbm, acc.at[pl.ds(s * RZ, RZ)])
        plsc.subcore_barrier()

        def body(j, carry):
            pltpu.sync_copy(ones_v, acc.at[dst_v.at[j]], add=True)
            return carry

        lax.fori_loop(40 * c, 40 + 39 * c, body, 0)
        plsc.subcore_barrier()
        pltpu.sync_copy(acc.at[pl.ds(s * RZ, RZ)],
                        out_hbm.at[c, pl.ds(s * RZ, RZ)])

    return deg_kernel


def _make_agg_kernel(split_features):
    @functools.partial(
        pl.kernel,
        out_type=jax.ShapeDtypeStruct((2, NP, 128), jnp.float32),
        mesh=_MESH,
        scratch_types=[
            pltpu.VMEM((NCHUNK, CH), jnp.int32),
            pltpu.VMEM((NCHUNK, CH), jnp.int32),
            pltpu.VMEM((CH, 128), jnp.float32),
            pltpu.VMEM_SHARED((NP, 128), jnp.float32),
            pltpu.SemaphoreType.DMA,
        ],
    )
    def agg_kernel(gidx_hbm, dst_hbm, g2v_hbm, zeros_hbm, out_hbm,
                   gidx_v, dst_v, rows_v, acc, sem):
        c = lax.axis_index("c")
        s = lax.axis_index("s")
        if split_features:
            pltpu.sync_copy(gidx_hbm.at[c, s], gidx_v)
        else:
            pltpu.sync_copy(gidx_hbm.at[s], gidx_v)
        pltpu.sync_copy(dst_hbm.at[s], dst_v)
        pltpu.sync_copy(zeros_hbm, acc.at[pl.ds(s * RZ, RZ)])
        plsc.subcore_barrier()

        def body(j, carry):
            pltpu.async_copy(g2v_hbm.at[gidx_v.at[j]], rows_v, sem).wait()
            pltpu.sync_copy(rows_v, acc.at[dst_v.at[j]], add=True)
            return carry

        if split_features:
            lax.fori_loop(0, NCHUNK, body, 0)
        else:
            lax.fori_loop(40 * c, 40 + 39 * c, body, 0)
        plsc.subcore_barrier()
        pltpu.sync_copy(acc.at[pl.ds(s * RZ, RZ)],
                        out_hbm.at[c, pl.ds(s * RZ, RZ)])

    return agg_kernel


_DEG_KERNEL = _make_deg_kernel()
_AGG_SPLIT = _make_agg_kernel(True)
_AGG_PART = _make_agg_kernel(False)


def _tc_first_body(x_b, w_b, d0_b, d1_b, g_b, dinv_b):
    deg = 1.0 + d0_b[:, 0:1] + d1_b[:, 0:1]
    dinv = lax.rsqrt(deg)
    dinv_b[...] = dinv
    g_b[...] = jnp.dot(x_b[...], w_b[...],
                       preferred_element_type=jnp.float32) * dinv


def _tc_first(x, w, d0, d1):
    f_in, f_out = w.shape
    return pl.pallas_call(
        _tc_first_body,
        grid=(GRID,),
        in_specs=[
            pl.BlockSpec((RB, f_in), lambda i: (i, 0)),
            pl.BlockSpec((f_in, f_out), lambda i: (0, 0)),
            pl.BlockSpec((RB, 128), lambda i: (i, 0)),
            pl.BlockSpec((RB, 128), lambda i: (i, 0)),
        ],
        out_specs=[
            pl.BlockSpec((RB, f_out), lambda i: (i, 0)),
            pl.BlockSpec((RB, 1), lambda i: (i, 0)),
        ],
        out_shape=[
            jax.ShapeDtypeStruct((NP, f_out), jnp.float32),
            jax.ShapeDtypeStruct((NP, 1), jnp.float32),
        ],
    )(x, w, d0, d1)


def _tc_mid_body(concat, scale_out, s0_b, s1_b, g_b, dinv_b, b_b, w_b, o_b):
    if concat:
        s = jnp.concatenate([s0_b[...], s1_b[...]], axis=1)
    else:
        s = s0_b[...] + s1_b[...]
    a = jnp.maximum((s + g_b[...]) * dinv_b[...] + b_b[...], 0.0)
    o = jnp.dot(a, w_b[...], preferred_element_type=jnp.float32)
    if scale_out:
        o = o * dinv_b[...]
    o_b[...] = o


def _tc_mid(s0, s1, g, dinv, b, w, concat, scale_out):
    f_in = g.shape[1]
    hw = f_in // 2 if concat else f_in
    f_out = w.shape[1]
    return pl.pallas_call(
        functools.partial(_tc_mid_body, concat, scale_out),
        grid=(GRID,),
        in_specs=[
            pl.BlockSpec((RB, hw), lambda i: (i, 0)),
            pl.BlockSpec((RB, hw), lambda i: (i, 0)),
            pl.BlockSpec((RB, f_in), lambda i: (i, 0)),
            pl.BlockSpec((RB, 1), lambda i: (i, 0)),
            pl.BlockSpec((1, f_in), lambda i: (0, 0)),
            pl.BlockSpec((f_in, f_out), lambda i: (0, 0)),
        ],
        out_specs=pl.BlockSpec((RB, f_out), lambda i: (i, 0)),
        out_shape=jax.ShapeDtypeStruct((NP, f_out), jnp.float32),
    )(s0, s1, g, dinv, b, w)


def _tc_final_body(s0_b, s1_b, g_b, dinv_b, b_b, w_b, bl_b, o_b):
    s = s0_b[...] + s1_b[...]
    a = jnp.maximum((s + g_b[...]) * dinv_b[...] + b_b[...], 0.0)
    o_b[...] = jnp.dot(a, w_b[...],
                       preferred_element_type=jnp.float32) + bl_b[...]


def _tc_final(s0, s1, g, dinv, b_pad, w_pad, bl_pad):
    f_in = g.shape[1]
    f_out = w_pad.shape[1]
    return pl.pallas_call(
        _tc_final_body,
        grid=(GRID,),
        in_specs=[
            pl.BlockSpec((RB, f_in), lambda i: (i, 0)),
            pl.BlockSpec((RB, f_in), lambda i: (i, 0)),
            pl.BlockSpec((RB, f_in), lambda i: (i, 0)),
            pl.BlockSpec((RB, 1), lambda i: (i, 0)),
            pl.BlockSpec((1, f_in), lambda i: (0, 0)),
            pl.BlockSpec((f_in, f_out), lambda i: (0, 0)),
            pl.BlockSpec((1, f_out), lambda i: (0, 0)),
        ],
        out_specs=pl.BlockSpec((RB, f_out), lambda i: (i, 0)),
        out_shape=jax.ShapeDtypeStruct((NP, f_out), jnp.float32),
    )(s0, s1, g, dinv, b_pad, w_pad, bl_pad)


def kernel(x, edge_index, W1, b1, W2, b2, W3, b3, Wl, bl):
    src = edge_index[0].astype(jnp.int32)
    dst = edge_index[1].astype(jnp.int32)
    pad = EP - E
    srcp = jnp.concatenate([src, jnp.zeros((pad,), jnp.int32)])
    dstp = jnp.concatenate([dst, jnp.full((pad,), N, jnp.int32)])
    gidx = (2 * srcp)[None, :] + jnp.array([[0], [1]], jnp.int32)
    gidx = gidx.reshape(2, NSUB, NCHUNK, CH)
    sidx = srcp.reshape(NSUB, NCHUNK, CH)
    dstr = dstp.reshape(NSUB, NCHUNK, CH)

    xp = jnp.pad(x, ((0, NP - N), (0, 0)))
    z128 = jnp.zeros((RZ, 128), jnp.float32)
    ones128 = jnp.ones((CH, 128), jnp.float32)
    sdeg = _DEG_KERNEL(dstr, ones128, z128)

    g1, dinv = _tc_first(xp, W1, sdeg[0], sdeg[1])
    s1 = _AGG_SPLIT(gidx, dstr, g1.reshape(2 * NP, 128), z128)
    g2 = _tc_mid(s1[0], s1[1], g1, dinv, b1.reshape(1, -1), W2,
                 concat=True, scale_out=True)
    s2 = _AGG_PART(sidx, dstr, g2, z128)
    w3_pad = jnp.pad(W3, ((0, 0), (0, 128 - W3.shape[1])))
    g3 = _tc_mid(s2[0], s2[1], g2, dinv, b2.reshape(1, -1), w3_pad,
                 concat=False, scale_out=True)
    s3 = _AGG_PART(sidx, dstr, g3, z128)
    b3_pad = jnp.pad(b3, (0, 128 - b3.shape[0])).reshape(1, -1)
    wl_pad = jnp.pad(Wl, ((0, 128 - Wl.shape[0]), (0, 128 - Wl.shape[1])))
    bl_pad = jnp.pad(bl, (0, 128 - bl.shape[0])).reshape(1, -1)
    out = _tc_final(s3[0], s3[1], g3, dinv, b3_pad, wl_pad, bl_pad)
    return out[:N, : bl.shape[0]]
